# baseline (speedup 1.0000x reference)
import functools
import math

import jax
import jax.numpy as jnp
from jax.experimental import pallas as pl
from jax.experimental.pallas import tpu as pltpu


def _cheb_body(g_ref, x_ref, w_ref, b_ref, out_ref, g8_ref, dsc_ref, *,
               np_steps, bm, n_orders, nb, c_out):
    f32 = jnp.float32
    bf16 = jnp.bfloat16
    i = pl.program_id(0)

    @pl.when(i < np_steps)
    def _prep():
        gf = g_ref[...]
        row0 = pl.multiple_of(i * bm, bm)
        g8_ref[pl.ds(row0, bm), :] = gf.astype(jnp.float8_e4m3fn)
        dsc_ref[pl.ds(row0, bm), :] = jax.lax.rsqrt(
            jnp.sum(gf, axis=1, keepdims=True))

    @pl.when(i >= np_steps)
    def _main():
        g = g8_ref[...]
        dsc = dsc_ref[...]
        dsc_up = dsc * 32.0
        dsc_dn = dsc * (1.0 / 32.0)

        def s_matvec(xv):
            xs = (dsc_up * xv).astype(jnp.float8_e4m3fn)
            return dsc_dn * jnp.dot(g, xs, preferred_element_type=f32)

        x0 = jnp.concatenate([x_ref[b] for b in range(nb)], axis=1)
        x1 = x0 - s_matvec(x0)
        xcat = [x0.astype(bf16), x1.astype(bf16)]
        xm2, xm1 = x0, x1
        for k in range(2, n_orders):
            xk = 2.0 * (xm1 - s_matvec(xm1)) - xm2
            xcat.append(xk.astype(bf16))
            xm2, xm1 = xm1, xk

        out = jnp.dot(jnp.concatenate(xcat, axis=1), w_ref[...],
                      preferred_element_type=f32) + b_ref[...]

        for b in range(nb):
            out_ref[b] = out[:, b * c_out:(b + 1) * c_out]


def kernel(inputs, graph, weight, bias):
    f32 = jnp.float32
    bf16 = jnp.bfloat16

    x = jnp.asarray(inputs, f32)
    batch, n, c_in = x.shape
    w = jnp.asarray(weight, f32)[:, 0]
    n_orders, _, c_out = w.shape
    b_vec = jnp.asarray(bias, f32).reshape(1, c_out)
    g = jnp.asarray(graph, f32)

    nb = batch // 2
    w_bd = jnp.einsum('be,kcd->kbced', jnp.eye(nb, dtype=f32), w)
    w_bd = w_bd.reshape(n_orders * nb * c_in, nb * c_out).astype(bf16)
    b_bd = jnp.tile(b_vec, (1, nb))

    bm = math.gcd(n, 1024)
    np_steps = n // bm

    out = pl.pallas_call(
        functools.partial(_cheb_body, np_steps=np_steps, bm=bm,
                          n_orders=n_orders, nb=nb, c_out=c_out),
        out_shape=jax.ShapeDtypeStruct((batch, n, c_out), f32),
        grid=(np_steps + 2,),
        in_specs=[
            pl.BlockSpec((bm, n),
                         lambda i: (jnp.minimum(i, np_steps - 1), 0)),
            pl.BlockSpec((nb, n, c_in),
                         lambda i: (jnp.maximum(i - np_steps, 0), 0, 0)),
            pl.BlockSpec((n_orders * nb * c_in, nb * c_out),
                         lambda i: (0, 0)),
            pl.BlockSpec((1, nb * c_out), lambda i: (0, 0)),
        ],
        out_specs=pl.BlockSpec((nb, n, c_out),
                               lambda i: (jnp.maximum(i - np_steps, 0),
                                          0, 0)),
        scratch_shapes=[
            pltpu.VMEM((n, n), jnp.float8_e4m3fn),
            pltpu.VMEM((n, 1), f32),
        ],
        compiler_params=pltpu.CompilerParams(
            dimension_semantics=("arbitrary",),
            vmem_limit_bytes=56 * 1024 * 1024,
        ),
    )(g, x, w_bd, b_bd)

    return out
